# Initial kernel scaffold; baseline (speedup 1.0000x reference)
import jax
import jax.numpy as jnp
from jax.experimental import pallas as pl


def kernel(x, input_pts, params):
    raise NotImplementedError("write your pallas kernel here")


# speedup vs baseline: 3.5431x; 3.5431x over previous
import functools

import jax
import jax.numpy as jnp
import numpy as np
from jax.experimental import pallas as pl

_NC = 27
_DIM = 3
_EPS = 1e-5
_MASKVAL = 1e30


def _conv_body(x_ref, pts_ref, ptsT_ref, q_ref, cf_ref, l1w_ref, l1b_ref,
               l2w_ref, l2b_ref, l3w_ref, l3b_ref, e_ref, wr_ref, b_ref,
               o_ref, *, K):
    x = x_ref[0]
    pts = pts_ref[0]
    ptsT = ptsT_ref[0]
    q = q_ref[0]
    bm = q.shape[0]
    n = pts.shape[0]
    cin = x.shape[1]

    d2 = jnp.zeros((bm, n), jnp.float32)
    for d in range(_DIM):
        diff = q[:, d:d + 1] - ptsT[d:d + 1, :]
        d2 = d2 + diff * diff

    iota = jax.lax.broadcasted_iota(jnp.int32, (bm, n), 1)
    g = jnp.concatenate([pts, x], axis=1)

    rels = []
    feats = []
    r2max = None
    for k in range(K):
        m = jnp.min(d2, axis=1, keepdims=True)
        eq = d2 <= m
        first = jnp.min(jnp.where(eq, iota, n), axis=1, keepdims=True)
        sel = iota == first
        sf = sel.astype(jnp.float32)
        nf = jnp.dot(sf, g, preferred_element_type=jnp.float32,
                     precision=jax.lax.Precision.HIGHEST)
        rel = nf[:, :_DIM] - q
        r2 = jnp.sum(rel * rel, axis=1, keepdims=True)
        r2max = r2 if r2max is None else jnp.maximum(r2max, r2)
        rels.append(rel)
        feats.append(nf[:, _DIM:])
        d2 = jnp.where(sel, _MASKVAL, d2)

    maxi = jnp.sqrt(r2max)
    maxi = jnp.where(maxi == 0.0, 1.0, maxi)

    cf = cf_ref[0]
    l1w = l1w_ref[0]
    l1b = l1b_ref[0]
    l2w = l2w_ref[0]
    l2b = l2b_ref[0]
    l3w = l3w_ref[0]
    l3b = l3b_ref[0]
    e = e_ref[0]
    wr = wr_ref[0]

    acc = jnp.zeros((bm, _NC * cin), jnp.float32)
    for k in range(K):
        reln = rels[k] / maxi
        rep = jnp.concatenate(
            [jnp.broadcast_to(reln[:, d:d + 1], (bm, _NC)) for d in range(_DIM)],
            axis=1)
        dists = rep - cf
        h = jnp.maximum(jnp.dot(dists, l1w, preferred_element_type=jnp.float32) + l1b, 0.0)
        h = jnp.maximum(jnp.dot(h, l2w, preferred_element_type=jnp.float32) + l2b, 0.0)
        h = jnp.maximum(jnp.dot(h, l3w, preferred_element_type=jnp.float32) + l3b, 0.0)
        if cin > 1:
            hb = h.astype(jnp.bfloat16).astype(jnp.float32)
            fb = feats[k].astype(jnp.bfloat16).astype(jnp.float32)
        else:
            hb = h
            fb = feats[k]
        hrep = jnp.dot(hb, e, preferred_element_type=jnp.float32,
                       precision=jax.lax.Precision.HIGHEST)
        ftile = jnp.concatenate([fb] * _NC, axis=1)
        acc = acc + hrep * ftile

    out = jnp.dot(acc, wr, preferred_element_type=jnp.float32) / K + b_ref[0]
    o_ref[0] = out


def _ptconv(x, pts, q, K, p):
    B, N, cin = x.shape
    M = q.shape[1]
    cout = p["W"].shape[2]
    bm = min(M, 256)

    ptsT = jnp.swapaxes(pts, 1, 2)
    cf = p["centers"].reshape(1, 1, _DIM * _NC)
    e = np.kron(np.eye(_NC, dtype=np.float32), np.ones((1, cin), np.float32))
    e = jnp.asarray(e)[None]
    wr = p["W"].transpose(1, 0, 2).reshape(1, _NC * cin, cout)
    l1w = p["l1w"][None]
    l1b = p["l1b"].reshape(1, 1, -1)
    l2w = p["l2w"][None]
    l2b = p["l2b"].reshape(1, 1, -1)
    l3w = p["l3w"][None]
    l3b = p["l3b"].reshape(1, 1, -1)
    bias = p["b"].reshape(1, 1, cout)

    def fixed(shape):
        nd = len(shape)
        return pl.BlockSpec(shape, lambda b, mb: (0,) * nd)

    grid = (B, M // bm)
    return pl.pallas_call(
        functools.partial(_conv_body, K=K),
        grid=grid,
        in_specs=[
            pl.BlockSpec((1, N, cin), lambda b, mb: (b, 0, 0)),
            pl.BlockSpec((1, N, _DIM), lambda b, mb: (b, 0, 0)),
            pl.BlockSpec((1, _DIM, N), lambda b, mb: (b, 0, 0)),
            pl.BlockSpec((1, bm, _DIM), lambda b, mb: (b, mb, 0)),
            fixed(cf.shape),
            fixed(l1w.shape),
            fixed(l1b.shape),
            fixed(l2w.shape),
            fixed(l2b.shape),
            fixed(l3w.shape),
            fixed(l3b.shape),
            fixed(e.shape),
            fixed(wr.shape),
            fixed(bias.shape),
        ],
        out_specs=pl.BlockSpec((1, bm, cout), lambda b, mb: (b, mb, 0)),
        out_shape=jax.ShapeDtypeStruct((B, M, cout), jnp.float32),
    )(x, pts, ptsT, q, cf, l1w, l1b, l2w, l2b, l3w, l3b, e, wr, bias)


def _bn_body(x_ref, g_ref, b_ref, o_ref):
    x = x_ref[...]
    m = jnp.mean(x, axis=0, keepdims=True)
    xc = x - m
    v = jnp.mean(xc * xc, axis=0, keepdims=True)
    y = g_ref[...] * xc / jnp.sqrt(v + _EPS) + b_ref[...]
    o_ref[...] = jnp.maximum(y, 0.0)


def _bn_relu(x, bnp):
    B, M, C = x.shape
    xf = x.reshape(B * M, C)
    out = pl.pallas_call(
        _bn_body,
        out_shape=jax.ShapeDtypeStruct((B * M, C), jnp.float32),
    )(xf, bnp["g"].reshape(1, C), bnp["b"].reshape(1, C))
    return out.reshape(B, M, C)


def _fc_body(x_ref, w_ref, b_ref, o_ref):
    o_ref[...] = (jnp.dot(x_ref[...], w_ref[...],
                          preferred_element_type=jnp.float32) + b_ref[...])


def kernel(x, input_pts, params):
    p1 = input_pts[:, ::2, :]
    p2 = input_pts[:, ::8, :]
    p3 = input_pts[:, ::32, :]
    p4 = input_pts[:, ::128, :]

    x1 = _ptconv(x, input_pts, p1, 16, params["cv1"])
    x1 = _bn_relu(x1, params["bn1"])
    x2 = _ptconv(x1, p1, p2, 16, params["cv2"])
    x2 = _bn_relu(x2, params["bn2"])
    x3 = _ptconv(x2, p2, p3, 8, params["cv3"])
    x3 = _bn_relu(x3, params["bn3"])
    x4 = _ptconv(x3, p3, p4, 8, params["cv4"])
    x4 = _bn_relu(x4, params["bn4"])
    p5 = input_pts[:, ::256, :]
    x5 = _ptconv(x4, p4, p5, 4, params["cv5"])
    x5 = _bn_relu(x5, params["bn5"])

    B = x5.shape[0]
    xflat = x5.reshape(B, -1)
    fcw = params["fc"]["W"]
    fcb = params["fc"]["b"].reshape(1, -1)
    return pl.pallas_call(
        _fc_body,
        out_shape=jax.ShapeDtypeStruct((B, fcw.shape[1]), jnp.float32),
    )(xflat, fcw, fcb)
